# baseline (speedup 1.0000x reference)
import functools
import math

import jax
import jax.numpy as jnp
from jax import lax
from jax.experimental import pallas as pl
from jax.experimental.pallas import tpu as pltpu
from jax.experimental.pallas import tpu_sc as plsc

D_MODEL = 512
SCALE = math.sqrt(D_MODEL ** 0.5)

_INFO = plsc.get_sparse_core_info()
_NC = _INFO.num_cores
_NS = _INFO.num_subcores
_L = _INFO.num_lanes
_NW = _NC * _NS

CHUNK = 64
NBUF = 3


def _make_gather(bsz, seq, d):
    n_rows = bsz * seq
    per_w = n_rows // _NW
    n_chunks = per_w // CHUNK
    w_per_row = seq // per_w
    mesh = plsc.VectorSubcoreMesh(core_axis_name="c", subcore_axis_name="s")

    @functools.partial(
        pl.kernel,
        mesh=mesh,
        out_type=jax.ShapeDtypeStruct((bsz, seq, d), jnp.float32),
        scratch_types=[
            pltpu.VMEM((per_w,), jnp.int32),
            pltpu.VMEM((NBUF * CHUNK, d), jnp.float32),
            pltpu.SemaphoreType.DMA,
            pltpu.SemaphoreType.DMA,
        ],
    )
    def k(idx_hbm, table_hbm, out_hbm, idx_v, buf, gsem, ssem):
        wid = lax.axis_index("s") * _NC + lax.axis_index("c")
        row = wid // w_per_row
        off = (wid % w_per_row) * per_w
        pltpu.sync_copy(idx_hbm.at[row, pl.ds(off, per_w)], idx_v)

        def g_desc(g, rb):
            return pltpu.make_async_copy(
                table_hbm.at[idx_v.at[pl.ds(g * CHUNK, CHUNK)]],
                buf.at[pl.ds(rb, CHUNK)], gsem)

        def s_desc(g, rb):
            return pltpu.make_async_copy(
                buf.at[pl.ds(rb, CHUNK)],
                out_hbm.at[row, pl.ds(off + g * CHUNK, CHUNK)], ssem)

        for g0 in range(min(NBUF, n_chunks)):
            g_desc(g0, g0 * CHUNK).start()

        @pl.loop(0, n_chunks)
        def _steady(g):
            rb = lax.rem(g, NBUF) * CHUNK
            g_desc(g, rb).wait()

            @pl.loop(rb, rb + CHUNK, step=2)
            def _rows(r):
                for rr in range(2):
                    for c in range(d // _L):
                        sl = pl.ds(c * _L, _L)
                        buf[r + rr, sl] = buf[r + rr, sl] * SCALE

            s_desc(g, rb).start()
            nxt = g + NBUF - 1

            @pl.when(jnp.logical_and(g >= 1, nxt < n_chunks))
            def _refill():
                pb = lax.rem(g - 1, NBUF) * CHUNK
                s_desc(g - 1, pb).wait()
                g_desc(nxt, pb).start()

        @pl.loop(max(0, n_chunks - NBUF), n_chunks)
        def _drain(g):
            rb = lax.rem(g, NBUF) * CHUNK
            s_desc(g, rb).wait()

    return k


def kernel(x, table):
    bsz, seq = x.shape
    d = table.shape[1]
    return _make_gather(bsz, seq, d)(x.astype(jnp.int32), table)


# speedup vs baseline: 1.1807x; 1.1807x over previous
import functools
import math

import jax
import jax.numpy as jnp
from jax import lax
from jax.experimental import pallas as pl
from jax.experimental.pallas import tpu as pltpu
from jax.experimental.pallas import tpu_sc as plsc

D_MODEL = 512
SCALE = math.sqrt(D_MODEL ** 0.5)

_INFO = plsc.get_sparse_core_info()
_NC = _INFO.num_cores
_NS = _INFO.num_subcores
_L = _INFO.num_lanes
_NW = _NC * _NS

CHUNK = 64
NBUF = 3


def _make_gather(bsz, seq, d):
    n_rows = bsz * seq
    per_w = n_rows // _NW
    n_chunks = per_w // CHUNK
    w_per_row = seq // per_w
    mesh = plsc.VectorSubcoreMesh(core_axis_name="c", subcore_axis_name="s")

    @functools.partial(
        pl.kernel,
        mesh=mesh,
        out_type=jax.ShapeDtypeStruct((bsz, seq, d), jnp.float32),
        scratch_types=[
            pltpu.VMEM((per_w,), jnp.int32),
            pltpu.VMEM((NBUF * CHUNK, d), jnp.float32),
            pltpu.SemaphoreType.DMA,
            pltpu.SemaphoreType.DMA,
        ],
    )
    def k(idx_hbm, table_hbm, out_hbm, idx_v, buf, gsem, ssem):
        wid = lax.axis_index("s") * _NC + lax.axis_index("c")
        row = wid // w_per_row
        off = (wid % w_per_row) * per_w
        pltpu.sync_copy(idx_hbm.at[row, pl.ds(off, per_w)], idx_v)

        def g_desc(g, rb):
            return pltpu.make_async_copy(
                table_hbm.at[idx_v.at[pl.ds(g * CHUNK, CHUNK)]],
                buf.at[pl.ds(rb, CHUNK)], gsem)

        HALF = CHUNK // 2

        def s_desc(g, rb, h):
            return pltpu.make_async_copy(
                buf.at[pl.ds(rb + h * HALF, HALF)],
                out_hbm.at[row, pl.ds(off + g * CHUNK + h * HALF, HALF)],
                ssem)

        for g0 in range(min(NBUF, n_chunks)):
            g_desc(g0, g0 * CHUNK).start()

        @pl.loop(0, n_chunks)
        def _steady(g):
            rb = lax.rem(g, NBUF) * CHUNK
            g_desc(g, rb).wait()

            for h in range(2):
                hb = rb + h * HALF

                def row_body(r, carry):
                    for c in range(d // _L):
                        sl = pl.ds(c * _L, _L)
                        buf[r, sl] = buf[r, sl] * SCALE
                    return carry
                lax.fori_loop(hb, hb + HALF, row_body, 0)

                s_desc(g, rb, h).start()

            nxt = g + NBUF - 1

            @pl.when(jnp.logical_and(g >= 1, nxt < n_chunks))
            def _refill():
                pb = lax.rem(g - 1, NBUF) * CHUNK
                s_desc(g - 1, pb, 0).wait()
                s_desc(g - 1, pb, 1).wait()
                g_desc(nxt, pb).start()

        @pl.loop(max(0, n_chunks - NBUF), n_chunks)
        def _drain(g):
            rb = lax.rem(g, NBUF) * CHUNK
            s_desc(g, rb, 0).wait()
            s_desc(g, rb, 1).wait()

    return k


def kernel(x, table):
    bsz, seq = x.shape
    d = table.shape[1]
    return _make_gather(bsz, seq, d)(x.astype(jnp.int32), table)
